# baseline (speedup 1.0000x reference)
import functools

import jax
import jax.numpy as jnp
from jax import lax
from jax.experimental import pallas as pl
from jax.experimental.pallas import tpu as pltpu
from jax.experimental.pallas import tpu_sc as plsc

N = 10000
E = 320000
D = 128
H = D // 2
NS = 16
CH = 125
ROWS = E // CH
RPTI = ROWS // NS
SPRC = 4
RPS = RPTI // SPRC
RPT = N // NS
NB = 4


def _gine_msg_body(x_hbm, src_hbm, dst_hbm, edge_hbm, out_hbm,
                   xtab, acc, sidx, didx, ebuf,
                   esem0, esem1, esem2, esem3, gsem0, gsem1, gsem2, gsem3,
                   ssem0, ssem1, ssem2, ssem3):
    c = lax.axis_index("c")
    s = lax.axis_index("s")
    col0 = c * H
    r0 = s * RPT
    pltpu.sync_copy(x_hbm.at[pl.ds(r0, RPT), pl.ds(col0, H)],
                    xtab.at[pl.ds(r0, RPT)])
    pltpu.sync_copy(x_hbm.at[pl.ds(r0, RPT), pl.ds(col0, H)],
                    acc.at[pl.ds(r0, RPT)])
    plsc.subcore_barrier()

    esems = (esem0, esem1, esem2, esem3)
    gsems = (gsem0, gsem1, gsem2, gsem3)
    ssems = (ssem0, ssem1, ssem2, ssem3)

    for sc in range(SPRC):
        rb = (s * SPRC + sc) * RPS
        eb0 = rb * CH

        def edge_cp(j, b):
            return pltpu.make_async_copy(
                edge_hbm.at[pl.ds(eb0 + j * CH, CH), pl.ds(col0, H)],
                ebuf.at[b], esems[b])

        def gather_cp(j, b):
            return pltpu.make_async_copy(
                xtab.at[sidx.at[j]], ebuf.at[b], gsems[b])

        def scatter_cp(j, b):
            return pltpu.make_async_copy(
                ebuf.at[b], acc.at[didx.at[j]], ssems[b])

        def chunk(j, b, first=False, prefetch=True, nxt=True):
            gather_cp(j, b).wait()

            if nxt:
                bn = (b + 1) % NB
                edge_cp(j + 1, bn).wait()
                gather_cp(j + 1, bn).start(add=True)

            @plsc.parallel_loop(0, CH, 1, unroll=5)
            def row(e):
                for v in range(H // 16):
                    sl = pl.ds(v * 16, 16)
                    ebuf[b, e, sl] = jnp.maximum(ebuf[b, e, sl], 0.0)

            scatter_cp(j, b).start(add=True)

            bo = (b + 3) % NB
            if not first:
                scatter_cp(j - 1, bo).wait()
            if prefetch:
                edge_cp(j + 3, bo).start()

        edge_cp(0, 0).start()
        edge_cp(1, 1).start()
        pltpu.sync_copy(src_hbm.at[pl.ds(rb, RPS)], sidx)
        pltpu.sync_copy(dst_hbm.at[pl.ds(rb, RPS)], didx)
        edge_cp(2, 2).start()
        edge_cp(0, 0).wait()
        gather_cp(0, 0).start(add=True)

        chunk(0, 0, first=True)

        def quad(m, carry):
            chunk(4 * m + 1, 1)
            chunk(4 * m + 2, 2)
            chunk(4 * m + 3, 3)
            chunk(4 * m + 4, 0)
            return carry

        Q = (RPS - 4) // 4
        lax.fori_loop(0, Q, quad, 0)
        for j in range(4 * Q + 1, RPS - 3):
            chunk(j, j % NB)
        for j in range(max(4 * Q + 1, RPS - 3), RPS - 1):
            chunk(j, j % NB, prefetch=False)
        chunk(RPS - 1, (RPS - 1) % NB, prefetch=False, nxt=False)
        scatter_cp(RPS - 1, (RPS - 1) % NB).wait()

    plsc.subcore_barrier()
    pltpu.sync_copy(acc.at[pl.ds(r0, RPT)],
                    out_hbm.at[pl.ds(r0, RPT), pl.ds(col0, H)])


_gine_msg = pl.kernel(
    _gine_msg_body,
    out_type=jax.ShapeDtypeStruct((N, D), jnp.float32),
    mesh=plsc.VectorSubcoreMesh(core_axis_name="c", subcore_axis_name="s"),
    compiler_params=pltpu.CompilerParams(use_tc_tiling_on_sc=False),
    scratch_types=[
        pltpu.VMEM_SHARED((N, H), jnp.float32),
        pltpu.VMEM_SHARED((N, H), jnp.float32),
        pltpu.VMEM((RPS, CH), jnp.int32),
        pltpu.VMEM((RPS, CH), jnp.int32),
        pltpu.VMEM((NB, CH, H), jnp.float32),
    ] + [pltpu.SemaphoreType.DMA] * 12,
)


BN = 2000


def _embed_body(z_ref, t_ref, emb_ref, w1_ref, wt_ref, b1_ref, w2_ref,
                b2_ref, o_ref):
    z = z_ref[...]
    onehot = (z == lax.broadcasted_iota(jnp.int32, (BN, 128), 1))
    node = jnp.dot(onehot.astype(jnp.float32), emb_ref[...],
                   preferred_element_type=jnp.float32)
    t = t_ref[...]
    h = node @ w1_ref[...] + t * wt_ref[...] + b1_ref[...]
    h = jnp.maximum(h, 0.0)
    o_ref[...] = h @ w2_ref[...] + b2_ref[...]


def _mlp_body(relu_out, y_ref, x_ref, w1_ref, b1_ref, w2_ref, b2_ref, o_ref):
    h = jnp.maximum(y_ref[...] @ w1_ref[...] + b1_ref[...], 0.0)
    o = h @ w2_ref[...] + b2_ref[...]
    if relu_out:
        o = jnp.maximum(o, 0.0)
    o_ref[...] = o + x_ref[...]


def _row_spec(w):
    return pl.BlockSpec((BN, w), lambda i: (i, 0))


def _full_spec(shape):
    return pl.BlockSpec(shape, lambda i: (0,) * len(shape))


def kernel(z, edge_index, edge_attr, t, emb, red_w1, red_b1, red_w2, red_b2,
           w1_0, b1_0, w2_0, b2_0, w1_1, b1_1, w2_1, b2_1,
           w1_2, b1_2, w2_2, b2_2):
    z2 = z.astype(jnp.int32).reshape(N, 1)
    t2 = t.reshape(N, 1)
    emb_p = jnp.pad(emb, ((0, 128 - emb.shape[0]), (0, 0)))
    rw1 = red_w1[:D]
    rwt = red_w1[D:D + 1]

    x = pl.pallas_call(
        _embed_body,
        grid=(N // BN,),
        in_specs=[
            _row_spec(1), _row_spec(1),
            _full_spec((128, D)), _full_spec((D, D)), _full_spec((1, D)),
            _full_spec((1, D)), _full_spec((D, D)), _full_spec((1, D)),
        ],
        out_specs=_row_spec(D),
        out_shape=jax.ShapeDtypeStruct((N, D), jnp.float32),
    )(z2, t2, emb_p, rw1, rwt, red_b1.reshape(1, D), red_w2,
      red_b2.reshape(1, D))

    src = edge_index[0].astype(jnp.int32).reshape(ROWS, CH)
    dst = edge_index[1].astype(jnp.int32).reshape(ROWS, CH)

    conv_ws = [(w1_0, b1_0, w2_0, b2_0), (w1_1, b1_1, w2_1, b2_1),
               (w1_2, b1_2, w2_2, b2_2)]
    for i, (w1, b1, w2, b2) in enumerate(conv_ws):
        y = _gine_msg(x, src, dst, edge_attr)
        x = pl.pallas_call(
            functools.partial(_mlp_body, i < 2),
            grid=(N // BN,),
            in_specs=[
                _row_spec(D), _row_spec(D),
                _full_spec((D, D)), _full_spec((1, D)),
                _full_spec((D, D)), _full_spec((1, D)),
            ],
            out_specs=_row_spec(D),
            out_shape=jax.ShapeDtypeStruct((N, D), jnp.float32),
        )(y, x, w1, b1.reshape(1, D), w2, b2.reshape(1, D))
    return x


# speedup vs baseline: 1.0455x; 1.0455x over previous
import functools

import jax
import jax.numpy as jnp
from jax import lax
from jax.experimental import pallas as pl
from jax.experimental.pallas import tpu as pltpu
from jax.experimental.pallas import tpu_sc as plsc

N = 10000
E = 320000
D = 128
H = D // 2
NS = 16
CH = 100
ROWS = E // CH
RPTI = ROWS // NS
SPRC = 2
RPS = RPTI // SPRC
RPT = N // NS
NB = 4


def _gine_msg_body(x_hbm, src_hbm, dst_hbm, edge_hbm, out_hbm,
                   xtab, acc, sidx, didx, ebuf,
                   esem0, esem1, esem2, esem3, gsem0, gsem1, gsem2, gsem3,
                   ssem0, ssem1, ssem2, ssem3):
    c = lax.axis_index("c")
    s = lax.axis_index("s")
    col0 = c * H
    r0 = s * RPT
    pltpu.sync_copy(x_hbm.at[pl.ds(r0, RPT), pl.ds(col0, H)],
                    xtab.at[pl.ds(r0, RPT)])
    pltpu.sync_copy(x_hbm.at[pl.ds(r0, RPT), pl.ds(col0, H)],
                    acc.at[pl.ds(r0, RPT)])
    plsc.subcore_barrier()

    esems = (esem0, esem1, esem2, esem3)
    gsems = (gsem0, gsem1, gsem2, gsem3)
    ssems = (ssem0, ssem1, ssem2, ssem3)

    for sc in range(SPRC):
        rb = (s * SPRC + sc) * RPS
        eb0 = rb * CH

        def edge_cp(j, b):
            return pltpu.make_async_copy(
                edge_hbm.at[pl.ds(eb0 + j * CH, CH), pl.ds(col0, H)],
                ebuf.at[b], esems[b])

        def gather_cp(j, b):
            return pltpu.make_async_copy(
                xtab.at[sidx.at[j]], ebuf.at[b], gsems[b])

        def scatter_cp(j, b):
            return pltpu.make_async_copy(
                ebuf.at[b], acc.at[didx.at[j]], ssems[b])

        def chunk(j, b, first=False, prefetch=True, nxt=True):
            gather_cp(j, b).wait()

            if nxt:
                bn = (b + 1) % NB
                edge_cp(j + 1, bn).wait()
                gather_cp(j + 1, bn).start(add=True)

            @plsc.parallel_loop(0, CH, 1, unroll=5)
            def row(e):
                for v in range(H // 16):
                    sl = pl.ds(v * 16, 16)
                    ebuf[b, e, sl] = jnp.maximum(ebuf[b, e, sl], 0.0)

            scatter_cp(j, b).start(add=True)

            bo = (b + 3) % NB
            if not first:
                scatter_cp(j - 1, bo).wait()
            if prefetch:
                edge_cp(j + 3, bo).start()

        edge_cp(0, 0).start()
        edge_cp(1, 1).start()
        pltpu.sync_copy(src_hbm.at[pl.ds(rb, RPS)], sidx)
        pltpu.sync_copy(dst_hbm.at[pl.ds(rb, RPS)], didx)
        edge_cp(2, 2).start()
        edge_cp(0, 0).wait()
        gather_cp(0, 0).start(add=True)

        chunk(0, 0, first=True)

        def quad(m, carry):
            chunk(4 * m + 1, 1)
            chunk(4 * m + 2, 2)
            chunk(4 * m + 3, 3)
            chunk(4 * m + 4, 0)
            return carry

        Q = (RPS - 4) // 4
        lax.fori_loop(0, Q, quad, 0)
        for j in range(4 * Q + 1, RPS - 3):
            chunk(j, j % NB)
        for j in range(max(4 * Q + 1, RPS - 3), RPS - 1):
            chunk(j, j % NB, prefetch=False)
        chunk(RPS - 1, (RPS - 1) % NB, prefetch=False, nxt=False)
        scatter_cp(RPS - 1, (RPS - 1) % NB).wait()

    plsc.subcore_barrier()
    pltpu.sync_copy(acc.at[pl.ds(r0, RPT)],
                    out_hbm.at[pl.ds(r0, RPT), pl.ds(col0, H)])


_gine_msg = pl.kernel(
    _gine_msg_body,
    out_type=jax.ShapeDtypeStruct((N, D), jnp.float32),
    mesh=plsc.VectorSubcoreMesh(core_axis_name="c", subcore_axis_name="s"),
    compiler_params=pltpu.CompilerParams(use_tc_tiling_on_sc=False),
    scratch_types=[
        pltpu.VMEM_SHARED((N, H), jnp.float32),
        pltpu.VMEM_SHARED((N, H), jnp.float32),
        pltpu.VMEM((RPS, CH), jnp.int32),
        pltpu.VMEM((RPS, CH), jnp.int32),
        pltpu.VMEM((NB, CH, H), jnp.float32),
    ] + [pltpu.SemaphoreType.DMA] * 12,
)


BN = 2000


def _embed_body(z_ref, t_ref, emb_ref, w1_ref, wt_ref, b1_ref, w2_ref,
                b2_ref, o_ref):
    z = z_ref[...]
    onehot = (z == lax.broadcasted_iota(jnp.int32, (BN, 128), 1))
    node = jnp.dot(onehot.astype(jnp.float32), emb_ref[...],
                   preferred_element_type=jnp.float32)
    t = t_ref[...]
    h = node @ w1_ref[...] + t * wt_ref[...] + b1_ref[...]
    h = jnp.maximum(h, 0.0)
    o_ref[...] = h @ w2_ref[...] + b2_ref[...]


def _mlp_body(relu_out, y_ref, x_ref, w1_ref, b1_ref, w2_ref, b2_ref, o_ref):
    h = jnp.maximum(y_ref[...] @ w1_ref[...] + b1_ref[...], 0.0)
    o = h @ w2_ref[...] + b2_ref[...]
    if relu_out:
        o = jnp.maximum(o, 0.0)
    o_ref[...] = o + x_ref[...]


def _row_spec(w):
    return pl.BlockSpec((BN, w), lambda i: (i, 0))


def _full_spec(shape):
    return pl.BlockSpec(shape, lambda i: (0,) * len(shape))


def kernel(z, edge_index, edge_attr, t, emb, red_w1, red_b1, red_w2, red_b2,
           w1_0, b1_0, w2_0, b2_0, w1_1, b1_1, w2_1, b2_1,
           w1_2, b1_2, w2_2, b2_2):
    z2 = z.astype(jnp.int32).reshape(N, 1)
    t2 = t.reshape(N, 1)
    emb_p = jnp.pad(emb, ((0, 128 - emb.shape[0]), (0, 0)))
    rw1 = red_w1[:D]
    rwt = red_w1[D:D + 1]

    x = pl.pallas_call(
        _embed_body,
        grid=(N // BN,),
        in_specs=[
            _row_spec(1), _row_spec(1),
            _full_spec((128, D)), _full_spec((D, D)), _full_spec((1, D)),
            _full_spec((1, D)), _full_spec((D, D)), _full_spec((1, D)),
        ],
        out_specs=_row_spec(D),
        out_shape=jax.ShapeDtypeStruct((N, D), jnp.float32),
    )(z2, t2, emb_p, rw1, rwt, red_b1.reshape(1, D), red_w2,
      red_b2.reshape(1, D))

    src = edge_index[0].astype(jnp.int32).reshape(ROWS, CH)
    dst = edge_index[1].astype(jnp.int32).reshape(ROWS, CH)

    conv_ws = [(w1_0, b1_0, w2_0, b2_0), (w1_1, b1_1, w2_1, b2_1),
               (w1_2, b1_2, w2_2, b2_2)]
    for i, (w1, b1, w2, b2) in enumerate(conv_ws):
        y = _gine_msg(x, src, dst, edge_attr)
        x = pl.pallas_call(
            functools.partial(_mlp_body, i < 2),
            grid=(N // BN,),
            in_specs=[
                _row_spec(D), _row_spec(D),
                _full_spec((D, D)), _full_spec((1, D)),
                _full_spec((D, D)), _full_spec((1, D)),
            ],
            out_specs=_row_spec(D),
            out_shape=jax.ShapeDtypeStruct((N, D), jnp.float32),
        )(y, x, w1, b1.reshape(1, D), w2, b2.reshape(1, D))
    return x


# speedup vs baseline: 1.1267x; 1.0777x over previous
import functools

import jax
import jax.numpy as jnp
from jax import lax
from jax.experimental import pallas as pl
from jax.experimental.pallas import tpu as pltpu
from jax.experimental.pallas import tpu_sc as plsc

N = 10000
E = 320000
D = 128
H = D // 2
NS = 16
CH = 80
ROWS = E // CH
RPTI = ROWS // NS
SPRC = 5
RPS = RPTI // SPRC
RPT = N // NS
NB = 6
GL = 2


def _gine_msg_body(x_hbm, src_hbm, dst_hbm, edge_hbm, out_hbm,
                   xtab, acc, sidx, didx, ebuf, *sems):
    c = lax.axis_index("c")
    s = lax.axis_index("s")
    col0 = c * H
    r0 = s * RPT
    pltpu.sync_copy(x_hbm.at[pl.ds(r0, RPT), pl.ds(col0, H)],
                    xtab.at[pl.ds(r0, RPT)])
    pltpu.sync_copy(x_hbm.at[pl.ds(r0, RPT), pl.ds(col0, H)],
                    acc.at[pl.ds(r0, RPT)])
    plsc.subcore_barrier()

    esems = sems[0:NB]
    gsems = sems[NB:NB + 3]
    ssems = sems[NB + 3:NB + 6]

    for sc in range(SPRC):
        rb = (s * SPRC + sc) * RPS
        eb0 = rb * CH

        def edge_cp(j, b):
            return pltpu.make_async_copy(
                edge_hbm.at[pl.ds(eb0 + j * CH, CH), pl.ds(col0, H)],
                ebuf.at[b], esems[b])

        def gather_cp(j, b):
            return pltpu.make_async_copy(
                xtab.at[sidx.at[j]], ebuf.at[b], gsems[b % 3])

        def scatter_cp(j, b):
            return pltpu.make_async_copy(
                ebuf.at[b], acc.at[didx.at[j]], ssems[b % 3])

        def chunk(j, b, first=False, steady=False):
            gather_cp(j, b).wait()

            if steady or j + GL <= RPS - 1:
                bg = (b + GL) % NB
                edge_cp(j + GL, bg).wait()
                gather_cp(j + GL, bg).start(add=True)

            @plsc.parallel_loop(0, CH, 1, unroll=4)
            def row(e):
                for v in range(H // 16):
                    sl = pl.ds(v * 16, 16)
                    ebuf[b, e, sl] = jnp.maximum(ebuf[b, e, sl], 0.0)

            scatter_cp(j, b).start(add=True)

            bo = (b + NB - 1) % NB
            if not first:
                scatter_cp(j - 1, bo).wait()
            if steady or j + NB - 1 <= RPS - 1:
                edge_cp(j + NB - 1, bo).start()

        edge_cp(0, 0).start()
        edge_cp(1, 1).start()
        pltpu.sync_copy(src_hbm.at[pl.ds(rb, RPS)], sidx)
        pltpu.sync_copy(dst_hbm.at[pl.ds(rb, RPS)], didx)
        for k in range(2, NB - 1):
            edge_cp(k, k).start()
        edge_cp(0, 0).wait()
        gather_cp(0, 0).start(add=True)
        edge_cp(1, 1).wait()
        gather_cp(1, 1).start(add=True)

        chunk(0, 0, first=True)

        def group(m, carry):
            for k in range(NB):
                chunk(NB * m + 1 + k, (1 + k) % NB, steady=True)
            return carry

        Q = (RPS - NB) // NB
        lax.fori_loop(0, Q, group, 0)
        for j in range(NB * Q + 1, RPS):
            chunk(j, j % NB)
        scatter_cp(RPS - 1, (RPS - 1) % NB).wait()

    plsc.subcore_barrier()
    pltpu.sync_copy(acc.at[pl.ds(r0, RPT)],
                    out_hbm.at[pl.ds(r0, RPT), pl.ds(col0, H)])


_gine_msg = pl.kernel(
    _gine_msg_body,
    out_type=jax.ShapeDtypeStruct((N, D), jnp.float32),
    mesh=plsc.VectorSubcoreMesh(core_axis_name="c", subcore_axis_name="s"),
    compiler_params=pltpu.CompilerParams(use_tc_tiling_on_sc=False),
    scratch_types=[
        pltpu.VMEM_SHARED((N, H), jnp.float32),
        pltpu.VMEM_SHARED((N, H), jnp.float32),
        pltpu.VMEM((RPS, CH), jnp.int32),
        pltpu.VMEM((RPS, CH), jnp.int32),
        pltpu.VMEM((NB, CH, H), jnp.float32),
    ] + [pltpu.SemaphoreType.DMA] * (NB + 6),
)


BN = 2000


def _embed_body(z_ref, t_ref, emb_ref, w1_ref, wt_ref, b1_ref, w2_ref,
                b2_ref, o_ref):
    z = z_ref[...]
    onehot = (z == lax.broadcasted_iota(jnp.int32, (BN, 128), 1))
    node = jnp.dot(onehot.astype(jnp.float32), emb_ref[...],
                   preferred_element_type=jnp.float32)
    t = t_ref[...]
    h = node @ w1_ref[...] + t * wt_ref[...] + b1_ref[...]
    h = jnp.maximum(h, 0.0)
    o_ref[...] = h @ w2_ref[...] + b2_ref[...]


def _mlp_body(relu_out, y_ref, x_ref, w1_ref, b1_ref, w2_ref, b2_ref, o_ref):
    h = jnp.maximum(y_ref[...] @ w1_ref[...] + b1_ref[...], 0.0)
    o = h @ w2_ref[...] + b2_ref[...]
    if relu_out:
        o = jnp.maximum(o, 0.0)
    o_ref[...] = o + x_ref[...]


def _row_spec(w):
    return pl.BlockSpec((BN, w), lambda i: (i, 0))


def _full_spec(shape):
    return pl.BlockSpec(shape, lambda i: (0,) * len(shape))


def kernel(z, edge_index, edge_attr, t, emb, red_w1, red_b1, red_w2, red_b2,
           w1_0, b1_0, w2_0, b2_0, w1_1, b1_1, w2_1, b2_1,
           w1_2, b1_2, w2_2, b2_2):
    z2 = z.astype(jnp.int32).reshape(N, 1)
    t2 = t.reshape(N, 1)
    emb_p = jnp.pad(emb, ((0, 128 - emb.shape[0]), (0, 0)))
    rw1 = red_w1[:D]
    rwt = red_w1[D:D + 1]

    x = pl.pallas_call(
        _embed_body,
        grid=(N // BN,),
        in_specs=[
            _row_spec(1), _row_spec(1),
            _full_spec((128, D)), _full_spec((D, D)), _full_spec((1, D)),
            _full_spec((1, D)), _full_spec((D, D)), _full_spec((1, D)),
        ],
        out_specs=_row_spec(D),
        out_shape=jax.ShapeDtypeStruct((N, D), jnp.float32),
    )(z2, t2, emb_p, rw1, rwt, red_b1.reshape(1, D), red_w2,
      red_b2.reshape(1, D))

    src = edge_index[0].astype(jnp.int32).reshape(ROWS, CH)
    dst = edge_index[1].astype(jnp.int32).reshape(ROWS, CH)

    conv_ws = [(w1_0, b1_0, w2_0, b2_0), (w1_1, b1_1, w2_1, b2_1),
               (w1_2, b1_2, w2_2, b2_2)]
    for i, (w1, b1, w2, b2) in enumerate(conv_ws):
        y = _gine_msg(x, src, dst, edge_attr)
        x = pl.pallas_call(
            functools.partial(_mlp_body, i < 2),
            grid=(N // BN,),
            in_specs=[
                _row_spec(D), _row_spec(D),
                _full_spec((D, D)), _full_spec((1, D)),
                _full_spec((D, D)), _full_spec((1, D)),
            ],
            out_specs=_row_spec(D),
            out_shape=jax.ShapeDtypeStruct((N, D), jnp.float32),
        )(y, x, w1, b1.reshape(1, D), w2, b2.reshape(1, D))
    return x
